# baseline (speedup 1.0000x reference)
import jax
import jax.numpy as jnp
from jax.experimental import pallas as pl
from jax.experimental.pallas import tpu as pltpu

B = 50
NPG = 200
P = 5
LP1 = 4
ROWS_PER_G = NPG * P * LP1
D = 128
H = 128
GPB = 2
STEPS = B // GPB


def _lin_relu(h, W_ref, b_ref):
    return jax.nn.relu(
        jnp.dot(h, W_ref[...], preferred_element_type=jnp.float32)
        + b_ref[...])


def _fused(x_ref,
           lc_W0, lc_b0, lc_W1, lc_b1,
           gc_W0, gc_b0, gc_W1, gc_b1,
           lm_W0, lm_b0, lm_W1, lm_b1,
           gm_W0, gm_b0, gm_W1, gm_b1,
           bs_W0, bs_b0,
           dec_W0, dec_b0, dec_W1r, dec_b1,
           out_ref, acc_ref):
    g = pl.program_id(0)

    h = _lin_relu(x_ref[...], lc_W0, lc_b0)
    h = _lin_relu(h, lc_W1, lc_b1)

    parts = []
    for p in range(GPB * P):
        base = p * NPG * LP1
        acc = h[base:base + NPG]
        for l in range(1, LP1):
            acc = acc + h[base + l * NPG: base + (l + 1) * NPG]
        parts.append(acc)
    c = jnp.concatenate(parts, axis=0)

    c = _lin_relu(c, gc_W0, gc_b0)
    c = _lin_relu(c, gc_W1, gc_b1)
    c = _lin_relu(c, lm_W0, lm_b0)
    c = _lin_relu(c, lm_W1, lm_b1)

    merged = []
    for gg in range(GPB):
        a = c[gg * P * NPG: gg * P * NPG + NPG]
        for p in range(1, P):
            a = a + c[(gg * P + p) * NPG:(gg * P + p + 1) * NPG]
        merged.append(a)
    acc_ref[pl.ds(g * GPB * NPG, GPB * NPG), :] = jnp.concatenate(
        merged, axis=0)

    @pl.when(g == STEPS - 1)
    def _tail():
        t = acc_ref[...]
        t = _lin_relu(t, gm_W0, gm_b0)
        t = _lin_relu(t, gm_W1, gm_b1)
        t = _lin_relu(t, bs_W0, bs_b0)
        pooled = jnp.concatenate(
            [jnp.sum(t[i * NPG:(i + 1) * NPG], axis=0, keepdims=True)
             for i in range(B)], axis=0)
        hd = _lin_relu(pooled, dec_W0, dec_b0)
        o = jnp.dot(hd, dec_W1r[...], preferred_element_type=jnp.float32)
        out_ref[...] = o + dec_b1[...]


def kernel(x, ptr, lc_W0, lc_b0, lc_W1, lc_b1, gc_W0, gc_b0, gc_W1, gc_b1,
           lm_W0, lm_b0, lm_W1, lm_b1, gm_W0, gm_b0, gm_W1, gm_b1,
           bs_W0, bs_b0, dec_W0, dec_b0, dec_W1, dec_b1):
    del ptr
    r2 = lambda v: v.reshape(1, -1)
    full = lambda s: pl.BlockSpec(s, lambda g: (0,) * len(s))

    weights = [
        lc_W0, r2(lc_b0), lc_W1, r2(lc_b1),
        gc_W0, r2(gc_b0), gc_W1, r2(gc_b1),
        lm_W0, r2(lm_b0), lm_W1, r2(lm_b1),
        gm_W0, r2(gm_b0), gm_W1, r2(gm_b1),
        bs_W0, r2(bs_b0),
        dec_W0, r2(dec_b0),
        jnp.pad(dec_W1, ((0, 0), (0, D - dec_W1.shape[1]))),
        dec_b1.reshape(1, 1),
    ]
    in_specs = [pl.BlockSpec((GPB * ROWS_PER_G, D), lambda g: (g, 0))]
    in_specs += [full(w.shape) for w in weights]

    out = pl.pallas_call(
        _fused,
        grid=(STEPS,),
        in_specs=in_specs,
        out_specs=full((B, D)),
        out_shape=jax.ShapeDtypeStruct((B, D), jnp.float32),
        scratch_shapes=[pltpu.VMEM((B * NPG, D), jnp.float32)],
        compiler_params=pltpu.CompilerParams(
            dimension_semantics=("arbitrary",)),
    )(x, *weights)
    return out[:, :1]


# speedup vs baseline: 1.0041x; 1.0041x over previous
import jax
import jax.numpy as jnp
from jax.experimental import pallas as pl
from jax.experimental.pallas import tpu as pltpu

B = 50
NPG = 200
P = 5
LP1 = 4
ROWS_PER_G = NPG * P * LP1
D = 128
H = 128
GPB = 2
STEPS = B // GPB


def _lin_relu(h, W_ref, b_ref):
    return jax.nn.relu(
        jnp.dot(h, W_ref[...], preferred_element_type=jnp.float32)
        + b_ref[...])


def _fused(x_ref,
           lc_W0, lc_b0, lc_W1, lc_b1,
           gc_W0, gc_b0, gc_W1, gc_b1,
           lm_W0, lm_b0, lm_W1, lm_b1,
           gm_W0, gm_b0, gm_W1, gm_b1,
           bs_W0, bs_b0,
           dec_W0, dec_b0, dec_W1r, dec_b1,
           out_ref, acc_ref):
    g = pl.program_id(0)

    h = _lin_relu(x_ref[...], lc_W0, lc_b0)
    h = _lin_relu(h, lc_W1, lc_b1)

    parts = []
    for p in range(GPB * P):
        base = p * NPG * LP1
        acc = h[base:base + NPG]
        for l in range(1, LP1):
            acc = acc + h[base + l * NPG: base + (l + 1) * NPG]
        parts.append(acc)
    c = jnp.concatenate(parts, axis=0)

    c = _lin_relu(c, gc_W0, gc_b0)
    c = _lin_relu(c, gc_W1, gc_b1)
    c = _lin_relu(c, lm_W0, lm_b0)
    c = _lin_relu(c, lm_W1, lm_b1)

    for gg in range(GPB):
        a = c[gg * P * NPG: gg * P * NPG + NPG]
        for p in range(1, P):
            a = a + c[(gg * P + p) * NPG:(gg * P + p + 1) * NPG]
        acc_ref[pl.ds((g * GPB + gg) * NPG, NPG), :] = a

    @pl.when(g == STEPS - 1)
    def _tail():
        t = acc_ref[...]
        t = _lin_relu(t, gm_W0, gm_b0)
        t = _lin_relu(t, gm_W1, gm_b1)
        t = _lin_relu(t, bs_W0, bs_b0)
        pooled = jnp.concatenate(
            [jnp.sum(t[i * NPG:(i + 1) * NPG], axis=0, keepdims=True)
             for i in range(B)], axis=0)
        hd = _lin_relu(pooled, dec_W0, dec_b0)
        o = jnp.dot(hd, dec_W1r[...], preferred_element_type=jnp.float32)
        out_ref[...] = o + dec_b1[...]


def kernel(x, ptr, lc_W0, lc_b0, lc_W1, lc_b1, gc_W0, gc_b0, gc_W1, gc_b1,
           lm_W0, lm_b0, lm_W1, lm_b1, gm_W0, gm_b0, gm_W1, gm_b1,
           bs_W0, bs_b0, dec_W0, dec_b0, dec_W1, dec_b1):
    del ptr
    r2 = lambda v: v.reshape(1, -1)
    full = lambda s: pl.BlockSpec(s, lambda g: (0,) * len(s))

    weights = [
        lc_W0, r2(lc_b0), lc_W1, r2(lc_b1),
        gc_W0, r2(gc_b0), gc_W1, r2(gc_b1),
        lm_W0, r2(lm_b0), lm_W1, r2(lm_b1),
        gm_W0, r2(gm_b0), gm_W1, r2(gm_b1),
        bs_W0, r2(bs_b0),
        dec_W0, r2(dec_b0),
        jnp.pad(dec_W1, ((0, 0), (0, D - dec_W1.shape[1]))),
        dec_b1.reshape(1, 1),
    ]
    in_specs = [pl.BlockSpec((GPB * ROWS_PER_G, D), lambda g: (g, 0))]
    in_specs += [full(w.shape) for w in weights]

    out = pl.pallas_call(
        _fused,
        grid=(STEPS,),
        in_specs=in_specs,
        out_specs=full((B, D)),
        out_shape=jax.ShapeDtypeStruct((B, D), jnp.float32),
        scratch_shapes=[pltpu.VMEM((B * NPG, D), jnp.float32)],
        compiler_params=pltpu.CompilerParams(
            dimension_semantics=("arbitrary",)),
    )(x, *weights)
    return out[:, :1]


# speedup vs baseline: 1.0500x; 1.0457x over previous
import jax
import jax.numpy as jnp
from jax.experimental import pallas as pl
from jax.experimental.pallas import tpu as pltpu

B = 50
NPG = 200
P = 5
LP1 = 4
ROWS_PER_G = NPG * P * LP1
D = 128
H = 128
GPB = 2
STEPS = B // GPB


def _lin_relu(h, W_ref, b_ref):
    return jax.nn.relu(
        jnp.dot(h, W_ref[...], preferred_element_type=jnp.float32)
        + b_ref[...])


def _fused(x_ref,
           lc_W0, lc_b0, lc_W1, lc_b1,
           gc_W0, gc_b0, gc_W1, gc_b1,
           lm_W0, lm_b0, lm_W1, lm_b1,
           gm_W0, gm_b0, gm_W1, gm_b1,
           bs_W0, bs_b0,
           dec_W0, dec_b0, dec_W1r, dec_b1,
           out_ref, acc_ref):
    g = pl.program_id(0)

    h = _lin_relu(x_ref[...], lc_W0, lc_b0)
    h = _lin_relu(h, lc_W1, lc_b1)

    parts = []
    for p in range(GPB * P):
        base = p * NPG * LP1
        acc = h[base:base + NPG]
        for l in range(1, LP1):
            acc = acc + h[base + l * NPG: base + (l + 1) * NPG]
        parts.append(acc)
    c = jnp.concatenate(parts, axis=0)

    c = _lin_relu(c, gc_W0, gc_b0)
    c = _lin_relu(c, gc_W1, gc_b1)
    c = _lin_relu(c, lm_W0, lm_b0)
    c = _lin_relu(c, lm_W1, lm_b1)

    for gg in range(GPB):
        a = c[gg * P * NPG: gg * P * NPG + NPG]
        for p in range(1, P):
            a = a + c[(gg * P + p) * NPG:(gg * P + p + 1) * NPG]
        acc_ref[pl.ds((g * GPB + gg) * NPG, NPG), :] = a

    @pl.when(g == STEPS - 1)
    def _tail():
        t = acc_ref[...]
        t = _lin_relu(t, gm_W0, gm_b0)
        t = _lin_relu(t, gm_W1, gm_b1)
        t = _lin_relu(t, bs_W0, bs_b0)
        pooled = jnp.concatenate(
            [jnp.sum(t[i * NPG:(i + 1) * NPG], axis=0, keepdims=True)
             for i in range(B)], axis=0)
        hd = _lin_relu(pooled, dec_W0, dec_b0)
        o = jnp.dot(hd, dec_W1r[...], preferred_element_type=jnp.float32)
        out_ref[...] = o + dec_b1[...]


def kernel(x, ptr, lc_W0, lc_b0, lc_W1, lc_b1, gc_W0, gc_b0, gc_W1, gc_b1,
           lm_W0, lm_b0, lm_W1, lm_b1, gm_W0, gm_b0, gm_W1, gm_b1,
           bs_W0, bs_b0, dec_W0, dec_b0, dec_W1, dec_b1):
    del ptr
    r2 = lambda v: v.reshape(1, -1)
    full = lambda s: pl.BlockSpec(s, lambda g: (0,) * len(s))

    weights = [
        lc_W0, r2(lc_b0), lc_W1, r2(lc_b1),
        gc_W0, r2(gc_b0), gc_W1, r2(gc_b1),
        lm_W0, r2(lm_b0), lm_W1, r2(lm_b1),
        gm_W0, r2(gm_b0), gm_W1, r2(gm_b1),
        bs_W0, r2(bs_b0),
        dec_W0, r2(dec_b0),
        jnp.pad(dec_W1, ((0, 0), (0, D - dec_W1.shape[1]))),
        dec_b1.reshape(1, 1),
    ]
    in_specs = [pl.BlockSpec((GPB * ROWS_PER_G, D), lambda g: (g, 0))]
    in_specs += [full(w.shape) for w in weights]

    out = pl.pallas_call(
        _fused,
        grid=(STEPS,),
        in_specs=in_specs,
        out_specs=full((B, D)),
        out_shape=jax.ShapeDtypeStruct((B, D), jnp.float32),
        scratch_shapes=[pltpu.VMEM((B * NPG, D), jnp.float32)],
        compiler_params=pltpu.CompilerParams(
            dimension_semantics=("arbitrary",),
            allow_input_fusion=[False] + [True] * 22),
    )(x, *weights)
    return out[:, :1]
